# Initial kernel scaffold; baseline (speedup 1.0000x reference)
import jax
import jax.numpy as jnp
from jax.experimental import pallas as pl


def kernel(x, weight, bias):
    raise NotImplementedError("write your pallas kernel here")


# speedup vs baseline: 1.5347x; 1.5347x over previous
import functools

import jax
import jax.numpy as jnp
from jax.experimental import pallas as pl

_DIM = 4096
_N_EXPERTS = 64
_TOPK = 8
_N_GROUPS = 8
_GROUP_SIZE = _N_EXPERTS // _N_GROUPS
_TOPK_GROUPS = 4
_ROUTE_SCALE = 2.5

_NEG = float("-inf")


def _router_kernel(x_ref, w_ref, b_ref, wout_ref, iout_ref):
    x = x_ref[...]
    w = w_ref[...]
    b = b_ref[...]

    logits = jax.lax.dot_general(
        x, w, (((1,), (1,)), ((), ())), preferred_element_type=jnp.float32)
    scores = jax.nn.sigmoid(logits + b)
    s = scores + b

    tb = x.shape[0]
    lane = jax.lax.broadcasted_iota(jnp.int32, (tb, _N_EXPERTS), 1)
    gid = lane // _GROUP_SIZE

    gs_cols = []
    for g in range(_N_GROUPS):
        sg = jnp.where(gid == g, s, _NEG)
        m1 = jnp.max(sg, axis=1, keepdims=True)
        l1 = jnp.min(jnp.where(sg == m1, lane, _N_EXPERTS), axis=1,
                     keepdims=True)
        m2 = jnp.max(jnp.where(lane == l1, _NEG, sg), axis=1, keepdims=True)
        gs_cols.append(m1 + m2)
    gscores = jnp.concatenate(gs_cols, axis=1)

    glane = jax.lax.broadcasted_iota(jnp.int32, (tb, _N_GROUPS), 1)
    sel = jnp.zeros((tb, _N_GROUPS), dtype=jnp.bool_)
    gtmp = gscores
    for _ in range(_TOPK_GROUPS):
        gm = jnp.max(gtmp, axis=1, keepdims=True)
        gl = jnp.min(jnp.where(gtmp == gm, glane, _N_GROUPS), axis=1,
                     keepdims=True)
        sel = sel | (glane == gl)
        gtmp = jnp.where(glane == gl, _NEG, gtmp)

    allowed = jnp.zeros((tb, _N_EXPERTS), dtype=jnp.bool_)
    for g in range(_N_GROUPS):
        allowed = allowed | ((gid == g) & sel[:, g:g + 1])

    sm = jnp.where(allowed, s, _NEG)
    idx_cols, w_cols = [], []
    for _ in range(_TOPK):
        m = jnp.max(sm, axis=1, keepdims=True)
        l = jnp.min(jnp.where(sm == m, lane, _N_EXPERTS), axis=1,
                    keepdims=True)
        hit = lane == l
        w_cols.append(jnp.max(jnp.where(hit, scores, _NEG), axis=1,
                              keepdims=True))
        idx_cols.append(l)
        sm = jnp.where(hit, _NEG, sm)

    idx = jnp.concatenate(idx_cols, axis=1)
    wts = jnp.concatenate(w_cols, axis=1)
    wts = wts * (_ROUTE_SCALE / jnp.sum(wts, axis=1, keepdims=True))

    wout_ref[...] = wts
    iout_ref[...] = idx


@functools.partial(jax.jit, static_argnames=())
def kernel(x, weight, bias):
    t = x.shape[0]
    tb = 256
    b2 = bias.reshape(1, _N_EXPERTS)
    wts, idx = pl.pallas_call(
        _router_kernel,
        grid=(t // tb,),
        in_specs=[
            pl.BlockSpec((tb, _DIM), lambda i: (i, 0)),
            pl.BlockSpec((_N_EXPERTS, _DIM), lambda i: (0, 0)),
            pl.BlockSpec((1, _N_EXPERTS), lambda i: (0, 0)),
        ],
        out_specs=[
            pl.BlockSpec((tb, _TOPK), lambda i: (i, 0)),
            pl.BlockSpec((tb, _TOPK), lambda i: (i, 0)),
        ],
        out_shape=[
            jax.ShapeDtypeStruct((t, _TOPK), jnp.float32),
            jax.ShapeDtypeStruct((t, _TOPK), jnp.int32),
        ],
    )(x, weight, b2)
    return wts, idx


# speedup vs baseline: 4.8373x; 3.1519x over previous
import functools

import jax
import jax.numpy as jnp
from jax.experimental import pallas as pl

_DIM = 4096
_N_EXPERTS = 64
_TOPK = 8
_N_GROUPS = 8
_GROUP_SIZE = _N_EXPERTS // _N_GROUPS
_TOPK_GROUPS = 4
_ROUTE_SCALE = 2.5

_NEG = float("-inf")


def _router_kernel(x_ref, w_ref, b_ref, wout_ref, iout_ref):
    x = x_ref[...]
    w = w_ref[...]
    b = b_ref[...]

    logits = jax.lax.dot_general(
        w, x, (((1,), (1,)), ((), ())), preferred_element_type=jnp.float32)
    scores = jax.nn.sigmoid(logits + b)
    s = scores + b

    tb = x.shape[0]

    row8 = jax.lax.broadcasted_iota(jnp.int32, (_GROUP_SIZE, tb), 0)
    gs_rows = []
    for g in range(_N_GROUPS):
        slab = s[g * _GROUP_SIZE:(g + 1) * _GROUP_SIZE, :]
        m1 = jnp.max(slab, axis=0, keepdims=True)
        r1 = jnp.min(jnp.where(slab == m1, row8, _GROUP_SIZE), axis=0,
                     keepdims=True)
        m2 = jnp.max(jnp.where(row8 == r1, _NEG, slab), axis=0,
                     keepdims=True)
        gs_rows.append(m1 + m2)
    gscores = jnp.concatenate(gs_rows, axis=0)

    grow = jax.lax.broadcasted_iota(jnp.int32, (_N_GROUPS, tb), 0)
    sel = jnp.zeros((_N_GROUPS, tb), dtype=jnp.bool_)
    gtmp = gscores
    for _ in range(_TOPK_GROUPS):
        gm = jnp.max(gtmp, axis=0, keepdims=True)
        gl = jnp.min(jnp.where(gtmp == gm, grow, _N_GROUPS), axis=0,
                     keepdims=True)
        sel = sel | (grow == gl)
        gtmp = jnp.where(grow == gl, _NEG, gtmp)

    sm_rows = []
    for g in range(_N_GROUPS):
        slab = s[g * _GROUP_SIZE:(g + 1) * _GROUP_SIZE, :]
        sm_rows.append(jnp.where(sel[g:g + 1, :], slab, _NEG))
    sm = jnp.concatenate(sm_rows, axis=0)

    row64 = jax.lax.broadcasted_iota(jnp.int32, (_N_EXPERTS, tb), 0)
    idx_rows, w_rows = [], []
    for _ in range(_TOPK):
        m = jnp.max(sm, axis=0, keepdims=True)
        l = jnp.min(jnp.where(sm == m, row64, _N_EXPERTS), axis=0,
                    keepdims=True)
        hit = row64 == l
        w_rows.append(jnp.max(jnp.where(hit, scores, _NEG), axis=0,
                              keepdims=True))
        idx_rows.append(l)
        sm = jnp.where(hit, _NEG, sm)

    idx = jnp.concatenate(idx_rows, axis=0)
    wts = jnp.concatenate(w_rows, axis=0)
    wts = wts * (_ROUTE_SCALE / jnp.sum(wts, axis=0, keepdims=True))

    wout_ref[...] = wts
    iout_ref[...] = idx


@functools.partial(jax.jit, static_argnames=())
def kernel(x, weight, bias):
    t = x.shape[0]
    tb = 256
    b2 = bias.reshape(_N_EXPERTS, 1)
    wts_t, idx_t = pl.pallas_call(
        _router_kernel,
        grid=(t // tb,),
        in_specs=[
            pl.BlockSpec((tb, _DIM), lambda i: (i, 0)),
            pl.BlockSpec((_N_EXPERTS, _DIM), lambda i: (0, 0)),
            pl.BlockSpec((_N_EXPERTS, 1), lambda i: (0, 0)),
        ],
        out_specs=[
            pl.BlockSpec((_TOPK, tb), lambda i: (0, i)),
            pl.BlockSpec((_TOPK, tb), lambda i: (0, i)),
        ],
        out_shape=[
            jax.ShapeDtypeStruct((_TOPK, t), jnp.float32),
            jax.ShapeDtypeStruct((_TOPK, t), jnp.int32),
        ],
    )(x, weight, b2)
    return wts_t.T, idx_t.T


# speedup vs baseline: 5.8593x; 1.2113x over previous
import functools

import jax
import jax.numpy as jnp
from jax.experimental import pallas as pl

_DIM = 4096
_N_EXPERTS = 64
_TOPK = 8
_N_GROUPS = 8
_GROUP_SIZE = _N_EXPERTS // _N_GROUPS
_TOPK_GROUPS = 4
_ROUTE_SCALE = 2.5

_NEG = float("-inf")


def _router_kernel(x_ref, w_ref, b_ref, wout_ref, iout_ref):
    x = x_ref[...]
    w = w_ref[...]
    b = b_ref[...]

    logits = jax.lax.dot_general(
        w, x, (((1,), (1,)), ((), ())), preferred_element_type=jnp.float32)
    scores = jax.nn.sigmoid(logits + b)
    s = scores + b

    tb = x.shape[0]

    row8 = jax.lax.broadcasted_iota(jnp.int32, (_GROUP_SIZE, tb), 0)
    gs_rows = []
    for g in range(_N_GROUPS):
        slab = s[g * _GROUP_SIZE:(g + 1) * _GROUP_SIZE, :]
        m1 = jnp.max(slab, axis=0, keepdims=True)
        r1 = jnp.min(jnp.where(slab == m1, row8, _GROUP_SIZE), axis=0,
                     keepdims=True)
        m2 = jnp.max(jnp.where(row8 == r1, _NEG, slab), axis=0,
                     keepdims=True)
        gs_rows.append(m1 + m2)
    gscores = jnp.concatenate(gs_rows, axis=0)

    grow = jax.lax.broadcasted_iota(jnp.int32, (_N_GROUPS, tb), 0)
    sel = jnp.zeros((_N_GROUPS, tb), dtype=jnp.bool_)
    gtmp = gscores
    for _ in range(_TOPK_GROUPS):
        gm = jnp.max(gtmp, axis=0, keepdims=True)
        gl = jnp.min(jnp.where(gtmp == gm, grow, _N_GROUPS), axis=0,
                     keepdims=True)
        sel = sel | (grow == gl)
        gtmp = jnp.where(grow == gl, _NEG, gtmp)

    sm_rows = []
    for g in range(_N_GROUPS):
        slab = s[g * _GROUP_SIZE:(g + 1) * _GROUP_SIZE, :]
        sm_rows.append(jnp.where(sel[g:g + 1, :], slab, _NEG))
    sm = jnp.concatenate(sm_rows, axis=0)

    row64 = jax.lax.broadcasted_iota(jnp.int32, (_N_EXPERTS, tb), 0)
    idx_rows, w_rows = [], []
    for _ in range(_TOPK):
        m = jnp.max(sm, axis=0, keepdims=True)
        l = jnp.min(jnp.where(sm == m, row64, _N_EXPERTS), axis=0,
                    keepdims=True)
        hit = row64 == l
        w_rows.append(jnp.max(jnp.where(hit, scores, _NEG), axis=0,
                              keepdims=True))
        idx_rows.append(l)
        sm = jnp.where(hit, _NEG, sm)

    idx = jnp.concatenate(idx_rows, axis=0)
    wts = jnp.concatenate(w_rows, axis=0)
    wts = wts * (_ROUTE_SCALE / jnp.sum(wts, axis=0, keepdims=True))

    wout_ref[...] = wts
    iout_ref[...] = idx


@functools.partial(jax.jit, static_argnames=())
def kernel(x, weight, bias):
    t = x.shape[0]
    tb = 512
    b2 = bias.reshape(_N_EXPERTS, 1)
    wts_t, idx_t = pl.pallas_call(
        _router_kernel,
        grid=(t // tb,),
        in_specs=[
            pl.BlockSpec((tb, _DIM), lambda i: (i, 0)),
            pl.BlockSpec((_N_EXPERTS, _DIM), lambda i: (0, 0)),
            pl.BlockSpec((_N_EXPERTS, 1), lambda i: (0, 0)),
        ],
        out_specs=[
            pl.BlockSpec((_TOPK, tb), lambda i: (0, i)),
            pl.BlockSpec((_TOPK, tb), lambda i: (0, i)),
        ],
        out_shape=[
            jax.ShapeDtypeStruct((_TOPK, t), jnp.float32),
            jax.ShapeDtypeStruct((_TOPK, t), jnp.int32),
        ],
    )(x, weight, b2)
    return wts_t.T, idx_t.T


# speedup vs baseline: 6.6766x; 1.1395x over previous
import functools

import jax
import jax.numpy as jnp
from jax.experimental import pallas as pl

_DIM = 4096
_N_EXPERTS = 64
_TOPK = 8
_N_GROUPS = 8
_GROUP_SIZE = _N_EXPERTS // _N_GROUPS
_TOPK_GROUPS = 4
_ROUTE_SCALE = 2.5

_NEG = float("-inf")


def _router_kernel(x_ref, w_ref, b_ref, wout_ref, iout_ref):
    x = x_ref[...]
    w = w_ref[...]
    b = b_ref[...]

    logits = jax.lax.dot_general(
        w, x, (((1,), (1,)), ((), ())), preferred_element_type=jnp.float32)
    scores = jax.nn.sigmoid(logits + b)
    s = scores + b

    tb = x.shape[0]

    row8 = jax.lax.broadcasted_iota(jnp.int32, (_GROUP_SIZE, tb), 0)
    gs_rows = []
    for g in range(_N_GROUPS):
        slab = s[g * _GROUP_SIZE:(g + 1) * _GROUP_SIZE, :]
        m1 = jnp.max(slab, axis=0, keepdims=True)
        r1 = jnp.min(jnp.where(slab == m1, row8, _GROUP_SIZE), axis=0,
                     keepdims=True)
        m2 = jnp.max(jnp.where(row8 == r1, _NEG, slab), axis=0,
                     keepdims=True)
        gs_rows.append(m1 + m2)
    gscores = jnp.concatenate(gs_rows, axis=0)

    grow = jax.lax.broadcasted_iota(jnp.int32, (_N_GROUPS, tb), 0)
    sel = jnp.zeros((_N_GROUPS, tb), dtype=jnp.bool_)
    gtmp = gscores
    for _ in range(_TOPK_GROUPS):
        gm = jnp.max(gtmp, axis=0, keepdims=True)
        gl = jnp.min(jnp.where(gtmp == gm, grow, _N_GROUPS), axis=0,
                     keepdims=True)
        sel = sel | (grow == gl)
        gtmp = jnp.where(grow == gl, _NEG, gtmp)

    sm_rows = []
    for g in range(_N_GROUPS):
        slab = s[g * _GROUP_SIZE:(g + 1) * _GROUP_SIZE, :]
        sm_rows.append(jnp.where(sel[g:g + 1, :], slab, _NEG))
    sm = jnp.concatenate(sm_rows, axis=0)

    row64 = jax.lax.broadcasted_iota(jnp.int32, (_N_EXPERTS, tb), 0)
    idx_rows, w_rows = [], []
    for _ in range(_TOPK):
        m = jnp.max(sm, axis=0, keepdims=True)
        l = jnp.min(jnp.where(sm == m, row64, _N_EXPERTS), axis=0,
                    keepdims=True)
        hit = row64 == l
        w_rows.append(jnp.max(jnp.where(hit, scores, _NEG), axis=0,
                              keepdims=True))
        idx_rows.append(l)
        sm = jnp.where(hit, _NEG, sm)

    idx = jnp.concatenate(idx_rows, axis=0)
    wts = jnp.concatenate(w_rows, axis=0)
    wts = wts * (_ROUTE_SCALE / jnp.sum(wts, axis=0, keepdims=True))

    wout_ref[...] = wts
    iout_ref[...] = idx


@functools.partial(jax.jit, static_argnames=())
def kernel(x, weight, bias):
    t = x.shape[0]
    tb = 1024
    b2 = bias.reshape(_N_EXPERTS, 1)
    wts_t, idx_t = pl.pallas_call(
        _router_kernel,
        grid=(t // tb,),
        in_specs=[
            pl.BlockSpec((tb, _DIM), lambda i: (i, 0)),
            pl.BlockSpec((_N_EXPERTS, _DIM), lambda i: (0, 0)),
            pl.BlockSpec((_N_EXPERTS, 1), lambda i: (0, 0)),
        ],
        out_specs=[
            pl.BlockSpec((_TOPK, tb), lambda i: (0, i)),
            pl.BlockSpec((_TOPK, tb), lambda i: (0, i)),
        ],
        out_shape=[
            jax.ShapeDtypeStruct((_TOPK, t), jnp.float32),
            jax.ShapeDtypeStruct((_TOPK, t), jnp.int32),
        ],
    )(x, weight, b2)
    return wts_t.T, idx_t.T
